# baseline (speedup 1.0000x reference)
import jax
import jax.numpy as jnp
from jax.experimental import pallas as pl

E = 16
DIM = 1024
INTER = 512
T = 32
A = 2
EPB = 2


def _moe_ffn_kernel(idx_ref, x_ref, w1_ref, w2_ref, out0_ref, out1_ref):
    g = pl.program_id(0)
    x = x_ref[...]
    c0 = None
    c1 = None
    for k in range(EPB):
        e = g * EPB + k
        h = jnp.dot(x, w1_ref[k], preferred_element_type=jnp.float32)
        gate = h[:, :INTER]
        up = h[:, INTER:]
        act = (gate * jax.nn.sigmoid(gate)) * up
        out_e = jnp.dot(act, w2_ref[k], preferred_element_type=jnp.float32)

        mask = idx_ref[...] == e
        k0 = jnp.where(mask[:, 0:1], out_e, 0.0)
        k1 = jnp.where(mask[:, 1:2], out_e, 0.0)
        c0 = k0 if c0 is None else c0 + k0
        c1 = k1 if c1 is None else c1 + k1

    @pl.when(g == 0)
    def _init():
        out0_ref[...] = c0
        out1_ref[...] = c1

    @pl.when(g != 0)
    def _accum():
        out0_ref[...] += c0
        out1_ref[...] += c1


def kernel(x, expert_indices, w1, w2):
    expert_indices = expert_indices.astype(jnp.int32)
    out0, out1 = pl.pallas_call(
        _moe_ffn_kernel,
        grid=(E // EPB,),
        in_specs=[
            pl.BlockSpec((T, A), lambda g: (0, 0)),
            pl.BlockSpec((T, DIM), lambda g: (0, 0)),
            pl.BlockSpec((EPB, DIM, 2 * INTER), lambda g: (g, 0, 0)),
            pl.BlockSpec((EPB, INTER, DIM), lambda g: (g, 0, 0)),
        ],
        out_specs=[
            pl.BlockSpec((T, DIM), lambda g: (0, 0)),
            pl.BlockSpec((T, DIM), lambda g: (0, 0)),
        ],
        out_shape=[
            jax.ShapeDtypeStruct((T, DIM), jnp.float32),
            jax.ShapeDtypeStruct((T, DIM), jnp.float32),
        ],
    )(expert_indices, x, w1, w2)
    return jnp.stack([out0, out1], axis=1)
